# baseline (speedup 1.0000x reference)
import jax
import jax.numpy as jnp
from jax import lax
from jax.experimental import pallas as pl
from jax.experimental.pallas import tpu as pltpu
from jax.experimental.pallas import tpu_sc as plsc

BATCH = 16384
FIELDS = 26
LATENT = 16
NFEAT = 1000000

NCORES = 2
NSUB = 16
NWORK = NCORES * NSUB

VC = 1024
NFULL = NFEAT // VC
NTAIL64 = 64
TAIL = NFEAT - NFULL * VC - NTAIL64
ITERS1 = (NFULL + NWORK - 1) // NWORK

RPW = BATCH // NWORK
CR = 64
NCH = RPW // CR
CI = CR * FIELDS
GW = 128
NG = CI // GW
IDX_ROWS = RPW * FIELDS // GW


def _relayout_body(fac_hbm, tail_hbm, out_hbm, buf0, buf1, st0, st1, sem0, sem1):
    wid = lax.axis_index("s") * NCORES + lax.axis_index("c")
    lanes16 = lax.iota(jnp.int32, 16) * 16

    def issue(c, nv, buf, sem):
        pltpu.async_copy(fac_hbm.at[:, :, pl.ds(c * VC, nv)],
                         buf.at[:, :, pl.ds(0, nv)], sem)

    def drain(c, nv, buf, sem):
        pltpu.make_async_copy(fac_hbm.at[:, :, pl.ds(c * VC, nv)],
                              buf.at[:, :, pl.ds(0, nv)], sem).wait()

    def transpose(c, nv_blocks, buf, stage, sem):
        @pl.loop(0, nv_blocks)
        def _(b):
            base = lanes16 + b * (16 * LATENT)
            for a in range(2):
                for s in range(8):
                    vals = buf[a, s, pl.ds(b * 16, 16)]
                    plsc.store_scatter(stage, [base + (a * 8 + s)], vals)
        pltpu.sync_copy(stage.at[pl.ds(0, nv_blocks * 16 * LATENT)],
                        out_hbm.at[pl.ds(c * VC * LATENT,
                                         nv_blocks * 16 * LATENT)])

    c0 = wid

    @pl.when(c0 < NFULL)
    def _():
        issue(c0, VC, buf0, sem0)

    @pl.loop(0, ITERS1, step=2)
    def _(it):
        ca = (it + 0) * NWORK + wid
        cb = (it + 1) * NWORK + wid
        cc = (it + 2) * NWORK + wid

        @pl.when(cb < NFULL)
        def _():
            issue(cb, VC, buf1, sem1)

        @pl.when(ca < NFULL)
        def _():
            drain(ca, VC, buf0, sem0)
            transpose(ca, VC // 16, buf0, st0, sem0)

        @pl.when(cc < NFULL)
        def _():
            issue(cc, VC, buf0, sem0)

        @pl.when(cb < NFULL)
        def _():
            drain(cb, VC, buf1, sem1)
            transpose(cb, VC // 16, buf1, st1, sem1)

    @pl.when(wid == NWORK - 1)
    def _():
        issue(NFULL, TAIL, buf0, sem0)
        drain(NFULL, TAIL, buf0, sem0)
        transpose(NFULL, TAIL // 16, buf0, st0, sem0)

    @pl.when(wid == NWORK - 2)
    def _():
        pltpu.sync_copy(tail_hbm, st0.at[pl.ds(0, NTAIL64 * LATENT)])
        pltpu.sync_copy(st0.at[pl.ds(0, NTAIL64 * LATENT)],
                        out_hbm.at[pl.ds((NFEAT - NTAIL64) * LATENT,
                                         NTAIL64 * LATENT)])


def _fm_body(x_hbm, bias_hbm, fac_hbm, out_hbm,
             idx_v, rows0, rows1, bias0, bias1, out_v, sem0, sem1):
    wid = lax.axis_index("s") * NCORES + lax.axis_index("c")

    pltpu.sync_copy(x_hbm.at[pl.ds(wid * IDX_ROWS, IDX_ROWS), :], idx_v)

    lanes = lax.iota(jnp.int32, 16)
    tail_mask = jnp.where(lanes < (FIELDS - 16), 1.0, 0.0).astype(jnp.float32)
    bias_1d = bias_hbm.at[0]

    def issue(ch, rows_v, bias_v, sem):
        for j in range(NG):
            irow = idx_v.at[ch * NG + j]
            pltpu.async_copy(fac_hbm.at[irow], rows_v.at[pl.ds(j * GW, GW), :], sem)
            pltpu.async_copy(bias_1d.at[irow],
                             bias_v.at[pl.ds(j * GW, GW)], sem)

    def drain(ch, rows_v, bias_v, sem):
        for j in range(NG):
            irow = idx_v.at[ch * NG + j]
            pltpu.make_async_copy(fac_hbm.at[irow],
                                  rows_v.at[pl.ds(j * GW, GW), :], sem).wait()
            pltpu.make_async_copy(bias_1d.at[irow],
                                  bias_v.at[pl.ds(j * GW, GW)], sem).wait()

    def compute(ch, rows_v, bias_v):
        @pl.loop(0, CR // 16)
        def _(g):
            def row_body(k, acc):
                base = (g * 16 + k) * FIELDS
                v = rows_v[base, :]
                s = v
                q = v * v
                for f in range(1, FIELDS):
                    v = rows_v[base + f, :]
                    s = s + v
                    q = q + v * v
                b1 = bias_v[pl.ds(base, 16)]
                b2 = bias_v[pl.ds(base + 16, 16)]
                tot = s * s - q + b1 + b2 * tail_mask
                return jnp.where(lanes == k, jnp.sum(tot), acc)

            acc = lax.fori_loop(0, 16, row_body,
                                jnp.zeros((16,), jnp.float32))
            out_v[pl.ds(ch * CR + g * 16, 16)] = acc

    issue(0, rows0, bias0, sem0)

    @pl.loop(0, NCH, step=2)
    def _(ch):
        issue(ch + 1, rows1, bias1, sem1)
        drain(ch, rows0, bias0, sem0)
        compute(ch, rows0, bias0)

        @pl.when(ch + 2 < NCH)
        def _():
            issue(ch + 2, rows0, bias0, sem0)

        drain(ch + 1, rows1, bias1, sem1)
        compute(ch + 1, rows1, bias1)

    pltpu.sync_copy(out_v, out_hbm.at[pl.ds(wid * RPW, RPW)])


@jax.jit
def _fm(x, emb_bias_w, emb_factor_w):
    x_idx = x.astype(jnp.int32).reshape(BATCH * FIELDS // GW, GW)
    fac_view = emb_factor_w.T.reshape(2, 8, NFEAT)
    tail_lin = emb_factor_w[NFEAT - NTAIL64:, :].reshape(NTAIL64 * LATENT)
    mesh = plsc.VectorSubcoreMesh(core_axis_name="c", subcore_axis_name="s")

    relayout = pl.kernel(
        _relayout_body,
        out_type=jax.ShapeDtypeStruct((NFEAT * LATENT,), jnp.float32),
        mesh=mesh,
        scratch_types=[
            pltpu.VMEM((2, 8, VC), jnp.float32),
            pltpu.VMEM((2, 8, VC), jnp.float32),
            pltpu.VMEM((VC * LATENT,), jnp.float32),
            pltpu.VMEM((VC * LATENT,), jnp.float32),
            pltpu.SemaphoreType.DMA,
            pltpu.SemaphoreType.DMA,
        ],
        compiler_params=pltpu.CompilerParams(needs_layout_passes=False,
                                             use_tc_tiling_on_sc=True),
    )
    fac_lin = relayout(fac_view, tail_lin).reshape(NFEAT, LATENT)

    run = pl.kernel(
        _fm_body,
        out_type=jax.ShapeDtypeStruct((BATCH,), jnp.float32),
        mesh=mesh,
        scratch_types=[
            pltpu.VMEM((IDX_ROWS, GW), jnp.int32),
            pltpu.VMEM((CI, LATENT), jnp.float32),
            pltpu.VMEM((CI, LATENT), jnp.float32),
            pltpu.VMEM((CI + 16,), jnp.float32),
            pltpu.VMEM((CI + 16,), jnp.float32),
            pltpu.VMEM((RPW,), jnp.float32),
            pltpu.SemaphoreType.DMA,
            pltpu.SemaphoreType.DMA,
        ],
        compiler_params=pltpu.CompilerParams(needs_layout_passes=False,
                                             use_tc_tiling_on_sc=False),
    )
    out = run(x_idx, emb_bias_w.T, fac_lin)
    return out.reshape(BATCH, 1)


def kernel(x, emb_bias_w, emb_factor_w):
    return _fm(x, emb_bias_w, emb_factor_w)


# speedup vs baseline: 1.5647x; 1.5647x over previous
import jax
import jax.numpy as jnp
from jax import lax
from jax.experimental import pallas as pl
from jax.experimental.pallas import tpu as pltpu
from jax.experimental.pallas import tpu_sc as plsc

BATCH = 16384
FIELDS = 26
LATENT = 16
NFEAT = 1000000

NCORES = 2
NSUB = 16
NWORK = NCORES * NSUB

VC = 1024
NFULL = NFEAT // VC
NTAIL64 = 64
TAIL = NFEAT - NFULL * VC - NTAIL64
ITERS1 = (NFULL + NWORK - 1) // NWORK

RPW = BATCH // NWORK
CR = 64
NCH = RPW // CR
CI = CR * FIELDS
GW = 128
NG = CI // GW
IDX_ROWS = RPW * FIELDS // GW


def _relayout_body(fac_hbm, tail_hbm, out_hbm, buf0, buf1, st0, st1, sem0, sem1):
    wid = lax.axis_index("s") * NCORES + lax.axis_index("c")
    lanes16 = lax.iota(jnp.int32, 16) * 16

    def issue(c, nv, buf, sem):
        pltpu.async_copy(fac_hbm.at[:, :, pl.ds(c * VC, nv)],
                         buf.at[:, :, pl.ds(0, nv)], sem)

    def drain(c, nv, buf, sem):
        pltpu.make_async_copy(fac_hbm.at[:, :, pl.ds(c * VC, nv)],
                              buf.at[:, :, pl.ds(0, nv)], sem).wait()

    def transpose(c, nv_blocks, buf, stage, sem):
        @pl.loop(0, nv_blocks)
        def _(b):
            base = lanes16 + b * (16 * LATENT)
            vals = [buf[a, s, pl.ds(b * 16, 16)]
                    for a in range(2) for s in range(8)]
            for k, v in enumerate(vals):
                plsc.store_scatter(stage, [base + k], v)
        pltpu.sync_copy(stage.at[pl.ds(0, nv_blocks * 16 * LATENT)],
                        out_hbm.at[pl.ds(c * VC * LATENT,
                                         nv_blocks * 16 * LATENT)])

    c0 = wid

    @pl.when(c0 < NFULL)
    def _():
        issue(c0, VC, buf0, sem0)

    @pl.loop(0, ITERS1, step=2)
    def _(it):
        ca = (it + 0) * NWORK + wid
        cb = (it + 1) * NWORK + wid
        cc = (it + 2) * NWORK + wid

        @pl.when(cb < NFULL)
        def _():
            issue(cb, VC, buf1, sem1)

        @pl.when(ca < NFULL)
        def _():
            drain(ca, VC, buf0, sem0)
            transpose(ca, VC // 16, buf0, st0, sem0)

        @pl.when(cc < NFULL)
        def _():
            issue(cc, VC, buf0, sem0)

        @pl.when(cb < NFULL)
        def _():
            drain(cb, VC, buf1, sem1)
            transpose(cb, VC // 16, buf1, st1, sem1)

    @pl.when(wid == NWORK - 1)
    def _():
        issue(NFULL, TAIL, buf0, sem0)
        drain(NFULL, TAIL, buf0, sem0)
        transpose(NFULL, TAIL // 16, buf0, st0, sem0)

    @pl.when(wid == NWORK - 2)
    def _():
        pltpu.sync_copy(tail_hbm, st0.at[pl.ds(0, NTAIL64 * LATENT)])
        pltpu.sync_copy(st0.at[pl.ds(0, NTAIL64 * LATENT)],
                        out_hbm.at[pl.ds((NFEAT - NTAIL64) * LATENT,
                                         NTAIL64 * LATENT)])


def _fm_body(x_hbm, bias_hbm, fac_hbm, out_hbm,
             idx_v, rows0, rows1, bias0, bias1, out_v, sem0, sem1):
    wid = lax.axis_index("s") * NCORES + lax.axis_index("c")

    pltpu.sync_copy(x_hbm.at[pl.ds(wid * IDX_ROWS, IDX_ROWS), :], idx_v)

    lanes = lax.iota(jnp.int32, 16)
    tail_mask = jnp.where(lanes < (FIELDS - 16), 1.0, 0.0).astype(jnp.float32)
    bias_1d = bias_hbm.at[0]

    def issue(ch, rows_v, bias_v, sem):
        for j in range(NG):
            irow = idx_v.at[ch * NG + j]
            pltpu.async_copy(fac_hbm.at[irow], rows_v.at[pl.ds(j * GW, GW), :], sem)
            pltpu.async_copy(bias_1d.at[irow],
                             bias_v.at[pl.ds(j * GW, GW)], sem)

    def drain(ch, rows_v, bias_v, sem):
        for j in range(NG):
            irow = idx_v.at[ch * NG + j]
            pltpu.make_async_copy(fac_hbm.at[irow],
                                  rows_v.at[pl.ds(j * GW, GW), :], sem).wait()
            pltpu.make_async_copy(bias_1d.at[irow],
                                  bias_v.at[pl.ds(j * GW, GW)], sem).wait()

    def compute(ch, rows_v, bias_v):
        @pl.loop(0, CR // 16)
        def _(g):
            def row_body(k, acc):
                base = (g * 16 + k) * FIELDS
                v = rows_v[base, :]
                s = v
                q = v * v
                for f in range(1, FIELDS):
                    v = rows_v[base + f, :]
                    s = s + v
                    q = q + v * v
                b1 = bias_v[pl.ds(base, 16)]
                b2 = bias_v[pl.ds(base + 16, 16)]
                tot = s * s - q + b1 + b2 * tail_mask
                return jnp.where(lanes == k, jnp.sum(tot), acc)

            acc = lax.fori_loop(0, 16, row_body,
                                jnp.zeros((16,), jnp.float32))
            out_v[pl.ds(ch * CR + g * 16, 16)] = acc

    issue(0, rows0, bias0, sem0)

    @pl.loop(0, NCH, step=2)
    def _(ch):
        issue(ch + 1, rows1, bias1, sem1)
        drain(ch, rows0, bias0, sem0)
        compute(ch, rows0, bias0)

        @pl.when(ch + 2 < NCH)
        def _():
            issue(ch + 2, rows0, bias0, sem0)

        drain(ch + 1, rows1, bias1, sem1)
        compute(ch + 1, rows1, bias1)

    pltpu.sync_copy(out_v, out_hbm.at[pl.ds(wid * RPW, RPW)])


@jax.jit
def _fm(x, emb_bias_w, emb_factor_w):
    x_idx = x.astype(jnp.int32).reshape(BATCH * FIELDS // GW, GW)
    fac_view = emb_factor_w.T.reshape(2, 8, NFEAT)
    tail_lin = emb_factor_w[NFEAT - NTAIL64:, :].reshape(NTAIL64 * LATENT)
    mesh = plsc.VectorSubcoreMesh(core_axis_name="c", subcore_axis_name="s")

    relayout = pl.kernel(
        _relayout_body,
        out_type=jax.ShapeDtypeStruct((NFEAT * LATENT,), jnp.float32),
        mesh=mesh,
        scratch_types=[
            pltpu.VMEM((2, 8, VC), jnp.float32),
            pltpu.VMEM((2, 8, VC), jnp.float32),
            pltpu.VMEM((VC * LATENT,), jnp.float32),
            pltpu.VMEM((VC * LATENT,), jnp.float32),
            pltpu.SemaphoreType.DMA,
            pltpu.SemaphoreType.DMA,
        ],
        compiler_params=pltpu.CompilerParams(needs_layout_passes=False,
                                             use_tc_tiling_on_sc=True),
    )
    fac_lin = relayout(fac_view, tail_lin).reshape(NFEAT, LATENT)

    run = pl.kernel(
        _fm_body,
        out_type=jax.ShapeDtypeStruct((BATCH,), jnp.float32),
        mesh=mesh,
        scratch_types=[
            pltpu.VMEM((IDX_ROWS, GW), jnp.int32),
            pltpu.VMEM((CI, LATENT), jnp.float32),
            pltpu.VMEM((CI, LATENT), jnp.float32),
            pltpu.VMEM((CI + 16,), jnp.float32),
            pltpu.VMEM((CI + 16,), jnp.float32),
            pltpu.VMEM((RPW,), jnp.float32),
            pltpu.SemaphoreType.DMA,
            pltpu.SemaphoreType.DMA,
        ],
        compiler_params=pltpu.CompilerParams(needs_layout_passes=False,
                                             use_tc_tiling_on_sc=False),
    )
    out = run(x_idx, emb_bias_w.T, fac_lin)
    return out.reshape(BATCH, 1)


def kernel(x, emb_bias_w, emb_factor_w):
    return _fm(x, emb_bias_w, emb_factor_w)


# speedup vs baseline: 1.5677x; 1.0019x over previous
import jax
import jax.numpy as jnp
from jax import lax
from jax.experimental import pallas as pl
from jax.experimental.pallas import tpu as pltpu
from jax.experimental.pallas import tpu_sc as plsc

BATCH = 16384
FIELDS = 26
LATENT = 16
NFEAT = 1000000

NCORES = 2
NSUB = 16
NWORK = NCORES * NSUB

VC = 1024
NFULL = NFEAT // VC
NTAIL64 = 64
TAIL = NFEAT - NFULL * VC - NTAIL64
ITERS1 = (NFULL + NWORK - 1) // NWORK

RPW = BATCH // NWORK
CR = 64
NCH = RPW // CR
CI = CR * FIELDS
GW = 128
NG = CI // GW
IDX_ROWS = RPW * FIELDS // GW


def _relayout_body(fac_hbm, tail_hbm, out_hbm, buf0, buf1, st0, st1, sem0, sem1):
    wid = lax.axis_index("s") * NCORES + lax.axis_index("c")
    lanes16 = lax.iota(jnp.int32, 16) * 16

    def issue(c, nv, buf, sem):
        pltpu.async_copy(fac_hbm.at[:, :, pl.ds(c * VC, nv)],
                         buf.at[:, :, pl.ds(0, nv)], sem)

    def drain(c, nv, buf, sem):
        pltpu.make_async_copy(fac_hbm.at[:, :, pl.ds(c * VC, nv)],
                              buf.at[:, :, pl.ds(0, nv)], sem).wait()

    def transpose(c, nv_blocks, buf, stage, sem):
        @pl.loop(0, nv_blocks)
        def _(b):
            base = lanes16 + b * (16 * LATENT)
            vals = [buf[a, s, pl.ds(b * 16, 16)]
                    for a in range(2) for s in range(8)]
            for k, v in enumerate(vals):
                plsc.store_scatter(stage, [base + k], v)
        pltpu.sync_copy(stage.at[pl.ds(0, nv_blocks * 16 * LATENT)],
                        out_hbm.at[pl.ds(c * VC * LATENT,
                                         nv_blocks * 16 * LATENT)])

    c0 = wid

    @pl.when(c0 < NFULL)
    def _():
        issue(c0, VC, buf0, sem0)

    @pl.loop(0, ITERS1, step=2)
    def _(it):
        ca = (it + 0) * NWORK + wid
        cb = (it + 1) * NWORK + wid
        cc = (it + 2) * NWORK + wid

        @pl.when(cb < NFULL)
        def _():
            issue(cb, VC, buf1, sem1)

        @pl.when(ca < NFULL)
        def _():
            drain(ca, VC, buf0, sem0)
            transpose(ca, VC // 16, buf0, st0, sem0)

        @pl.when(cc < NFULL)
        def _():
            issue(cc, VC, buf0, sem0)

        @pl.when(cb < NFULL)
        def _():
            drain(cb, VC, buf1, sem1)
            transpose(cb, VC // 16, buf1, st1, sem1)

    @pl.when(wid == NWORK - 1)
    def _():
        issue(NFULL, TAIL, buf0, sem0)
        drain(NFULL, TAIL, buf0, sem0)
        transpose(NFULL, TAIL // 16, buf0, st0, sem0)

    @pl.when(wid == NWORK - 2)
    def _():
        pltpu.sync_copy(tail_hbm, st0.at[pl.ds(0, NTAIL64 * LATENT)])
        pltpu.sync_copy(st0.at[pl.ds(0, NTAIL64 * LATENT)],
                        out_hbm.at[pl.ds((NFEAT - NTAIL64) * LATENT,
                                         NTAIL64 * LATENT)])


def _fm_body(x_hbm, bias_hbm, fac_hbm, out_hbm,
             idx_v, rows0, rows1, bias0, bias1, out_v, sem0, sem1):
    wid = lax.axis_index("s") * NCORES + lax.axis_index("c")

    pltpu.sync_copy(x_hbm.at[pl.ds(wid * IDX_ROWS, IDX_ROWS), :], idx_v)

    lanes = lax.iota(jnp.int32, 16)
    tail_mask = jnp.where(lanes < (FIELDS - 16), 1.0, 0.0).astype(jnp.float32)
    bias_1d = bias_hbm.at[0]

    def issue(ch, rows_v, bias_v, sem):
        for j in range(NG):
            irow = idx_v.at[ch * NG + j]
            pltpu.async_copy(fac_hbm.at[irow], rows_v.at[pl.ds(j * GW, GW), :], sem)
            pltpu.async_copy(bias_1d.at[irow],
                             bias_v.at[pl.ds(j * GW, GW)], sem)

    def drain(ch, rows_v, bias_v, sem):
        for j in range(NG):
            irow = idx_v.at[ch * NG + j]
            pltpu.make_async_copy(fac_hbm.at[irow],
                                  rows_v.at[pl.ds(j * GW, GW), :], sem).wait()
            pltpu.make_async_copy(bias_1d.at[irow],
                                  bias_v.at[pl.ds(j * GW, GW)], sem).wait()

    def compute(ch, rows_v, bias_v):
        @pl.loop(0, CR // 16)
        def _(g):
            def row_body(k, acc):
                base = (g * 16 + k) * FIELDS
                vs = [rows_v[base + f, :] for f in range(FIELDS)]
                ss = [vs[0], vs[1], vs[2], vs[3]]
                qs = [v * v for v in vs[:4]]
                for f in range(4, FIELDS):
                    ss[f % 4] = ss[f % 4] + vs[f]
                    qs[f % 4] = qs[f % 4] + vs[f] * vs[f]
                s = (ss[0] + ss[1]) + (ss[2] + ss[3])
                q = (qs[0] + qs[1]) + (qs[2] + qs[3])
                b1 = bias_v[pl.ds(base, 16)]
                b2 = bias_v[pl.ds(base + 16, 16)]
                tot = s * s - q + b1 + b2 * tail_mask
                return jnp.where(lanes == k, jnp.sum(tot), acc)

            acc = lax.fori_loop(0, 16, row_body,
                                jnp.zeros((16,), jnp.float32))
            out_v[pl.ds(ch * CR + g * 16, 16)] = acc

    issue(0, rows0, bias0, sem0)

    @pl.loop(0, NCH, step=2)
    def _(ch):
        issue(ch + 1, rows1, bias1, sem1)
        drain(ch, rows0, bias0, sem0)
        compute(ch, rows0, bias0)

        @pl.when(ch + 2 < NCH)
        def _():
            issue(ch + 2, rows0, bias0, sem0)

        drain(ch + 1, rows1, bias1, sem1)
        compute(ch + 1, rows1, bias1)

    pltpu.sync_copy(out_v, out_hbm.at[pl.ds(wid * RPW, RPW)])


@jax.jit
def _fm(x, emb_bias_w, emb_factor_w):
    x_idx = x.astype(jnp.int32).reshape(BATCH * FIELDS // GW, GW)
    fac_view = emb_factor_w.T.reshape(2, 8, NFEAT)
    tail_lin = emb_factor_w[NFEAT - NTAIL64:, :].reshape(NTAIL64 * LATENT)
    mesh = plsc.VectorSubcoreMesh(core_axis_name="c", subcore_axis_name="s")

    relayout = pl.kernel(
        _relayout_body,
        out_type=jax.ShapeDtypeStruct((NFEAT * LATENT,), jnp.float32),
        mesh=mesh,
        scratch_types=[
            pltpu.VMEM((2, 8, VC), jnp.float32),
            pltpu.VMEM((2, 8, VC), jnp.float32),
            pltpu.VMEM((VC * LATENT,), jnp.float32),
            pltpu.VMEM((VC * LATENT,), jnp.float32),
            pltpu.SemaphoreType.DMA,
            pltpu.SemaphoreType.DMA,
        ],
        compiler_params=pltpu.CompilerParams(needs_layout_passes=False,
                                             use_tc_tiling_on_sc=True),
    )
    fac_lin = relayout(fac_view, tail_lin).reshape(NFEAT, LATENT)

    run = pl.kernel(
        _fm_body,
        out_type=jax.ShapeDtypeStruct((BATCH,), jnp.float32),
        mesh=mesh,
        scratch_types=[
            pltpu.VMEM((IDX_ROWS, GW), jnp.int32),
            pltpu.VMEM((CI, LATENT), jnp.float32),
            pltpu.VMEM((CI, LATENT), jnp.float32),
            pltpu.VMEM((CI + 16,), jnp.float32),
            pltpu.VMEM((CI + 16,), jnp.float32),
            pltpu.VMEM((RPW,), jnp.float32),
            pltpu.SemaphoreType.DMA,
            pltpu.SemaphoreType.DMA,
        ],
        compiler_params=pltpu.CompilerParams(needs_layout_passes=False,
                                             use_tc_tiling_on_sc=False),
    )
    out = run(x_idx, emb_bias_w.T, fac_lin)
    return out.reshape(BATCH, 1)


def kernel(x, emb_bias_w, emb_factor_w):
    return _fm(x, emb_bias_w, emb_factor_w)


# speedup vs baseline: 1.5836x; 1.0102x over previous
import jax
import jax.numpy as jnp
from jax import lax
from jax.experimental import pallas as pl
from jax.experimental.pallas import tpu as pltpu
from jax.experimental.pallas import tpu_sc as plsc

BATCH = 16384
FIELDS = 26
LATENT = 16
NFEAT = 1000000

NCORES = 2
NSUB = 16
NWORK = NCORES * NSUB

NTAIL64 = 64
VC = 1536
NFULL = (NFEAT - NTAIL64) // VC
ITERS1 = (NFULL + NWORK - 1) // NWORK

RPW = BATCH // NWORK
CR = 64
NCH = RPW // CR
CI = CR * FIELDS
GW = 128
NG = CI // GW
IDX_ROWS = RPW * FIELDS // GW


def _relayout_body(fac_hbm, tail_hbm, out_hbm, buf0, buf1, st0, st1, sem0, sem1):
    wid = lax.axis_index("s") * NCORES + lax.axis_index("c")
    lanes16 = lax.iota(jnp.int32, 16) * 16

    def issue(c, nv, buf, sem):
        pltpu.async_copy(fac_hbm.at[:, :, pl.ds(c * VC, nv)],
                         buf.at[:, :, pl.ds(0, nv)], sem)

    def drain(c, nv, buf, sem):
        pltpu.make_async_copy(fac_hbm.at[:, :, pl.ds(c * VC, nv)],
                              buf.at[:, :, pl.ds(0, nv)], sem).wait()

    def transpose(c, nv_blocks, buf, stage, sem):
        @pl.loop(0, nv_blocks)
        def _(b):
            base = lanes16 + b * (16 * LATENT)
            vals = [buf[a, s, pl.ds(b * 16, 16)]
                    for a in range(2) for s in range(8)]
            for k, v in enumerate(vals):
                plsc.store_scatter(stage, [base + k], v)
        pltpu.sync_copy(stage.at[pl.ds(0, nv_blocks * 16 * LATENT)],
                        out_hbm.at[pl.ds(c * VC * LATENT,
                                         nv_blocks * 16 * LATENT)])

    c0 = wid

    @pl.when(c0 < NFULL)
    def _():
        issue(c0, VC, buf0, sem0)

    @pl.loop(0, ITERS1, step=2)
    def _(it):
        ca = (it + 0) * NWORK + wid
        cb = (it + 1) * NWORK + wid
        cc = (it + 2) * NWORK + wid

        @pl.when(cb < NFULL)
        def _():
            issue(cb, VC, buf1, sem1)

        @pl.when(ca < NFULL)
        def _():
            drain(ca, VC, buf0, sem0)
            transpose(ca, VC // 16, buf0, st0, sem0)

        @pl.when(cc < NFULL)
        def _():
            issue(cc, VC, buf0, sem0)

        @pl.when(cb < NFULL)
        def _():
            drain(cb, VC, buf1, sem1)
            transpose(cb, VC // 16, buf1, st1, sem1)

    @pl.when(wid == NWORK - 2)
    def _():
        pltpu.sync_copy(tail_hbm, st0.at[pl.ds(0, NTAIL64 * LATENT)])
        pltpu.sync_copy(st0.at[pl.ds(0, NTAIL64 * LATENT)],
                        out_hbm.at[pl.ds((NFEAT - NTAIL64) * LATENT,
                                         NTAIL64 * LATENT)])


def _fm_body(x_hbm, bias_hbm, fac_hbm, out_hbm,
             idx_v, rows0, rows1, bias0, bias1, out_v, sem0, sem1):
    wid = lax.axis_index("s") * NCORES + lax.axis_index("c")

    pltpu.sync_copy(x_hbm.at[pl.ds(wid * IDX_ROWS, IDX_ROWS), :], idx_v)

    lanes = lax.iota(jnp.int32, 16)
    tail_mask = jnp.where(lanes < (FIELDS - 16), 1.0, 0.0).astype(jnp.float32)
    bias_1d = bias_hbm.at[0]

    def issue(ch, rows_v, bias_v, sem):
        for j in range(NG):
            irow = idx_v.at[ch * NG + j]
            pltpu.async_copy(fac_hbm.at[irow], rows_v.at[pl.ds(j * GW, GW), :], sem)
            pltpu.async_copy(bias_1d.at[irow],
                             bias_v.at[pl.ds(j * GW, GW)], sem)

    def drain(ch, rows_v, bias_v, sem):
        for j in range(NG):
            irow = idx_v.at[ch * NG + j]
            pltpu.make_async_copy(fac_hbm.at[irow],
                                  rows_v.at[pl.ds(j * GW, GW), :], sem).wait()
            pltpu.make_async_copy(bias_1d.at[irow],
                                  bias_v.at[pl.ds(j * GW, GW)], sem).wait()

    def compute(ch, rows_v, bias_v):
        @pl.loop(0, CR // 16)
        def _(g):
            def row_body(k, acc):
                base = (g * 16 + k) * FIELDS
                vs = [rows_v[base + f, :] for f in range(FIELDS)]
                ss = [vs[0], vs[1], vs[2], vs[3]]
                qs = [v * v for v in vs[:4]]
                for f in range(4, FIELDS):
                    ss[f % 4] = ss[f % 4] + vs[f]
                    qs[f % 4] = qs[f % 4] + vs[f] * vs[f]
                s = (ss[0] + ss[1]) + (ss[2] + ss[3])
                q = (qs[0] + qs[1]) + (qs[2] + qs[3])
                b1 = bias_v[pl.ds(base, 16)]
                b2 = bias_v[pl.ds(base + 16, 16)]
                tot = s * s - q + b1 + b2 * tail_mask
                return jnp.where(lanes == k, jnp.sum(tot), acc)

            acc = lax.fori_loop(0, 16, row_body,
                                jnp.zeros((16,), jnp.float32))
            out_v[pl.ds(ch * CR + g * 16, 16)] = acc

    issue(0, rows0, bias0, sem0)

    @pl.loop(0, NCH, step=2)
    def _(ch):
        issue(ch + 1, rows1, bias1, sem1)
        drain(ch, rows0, bias0, sem0)
        compute(ch, rows0, bias0)

        @pl.when(ch + 2 < NCH)
        def _():
            issue(ch + 2, rows0, bias0, sem0)

        drain(ch + 1, rows1, bias1, sem1)
        compute(ch + 1, rows1, bias1)

    pltpu.sync_copy(out_v, out_hbm.at[pl.ds(wid * RPW, RPW)])


@jax.jit
def _fm(x, emb_bias_w, emb_factor_w):
    x_idx = x.astype(jnp.int32).reshape(BATCH * FIELDS // GW, GW)
    fac_view = emb_factor_w.T.reshape(2, 8, NFEAT)
    tail_lin = emb_factor_w[NFEAT - NTAIL64:, :].reshape(NTAIL64 * LATENT)
    mesh = plsc.VectorSubcoreMesh(core_axis_name="c", subcore_axis_name="s")

    relayout = pl.kernel(
        _relayout_body,
        out_type=jax.ShapeDtypeStruct((NFEAT * LATENT,), jnp.float32),
        mesh=mesh,
        scratch_types=[
            pltpu.VMEM((2, 8, VC), jnp.float32),
            pltpu.VMEM((2, 8, VC), jnp.float32),
            pltpu.VMEM((VC * LATENT,), jnp.float32),
            pltpu.VMEM((VC * LATENT,), jnp.float32),
            pltpu.SemaphoreType.DMA,
            pltpu.SemaphoreType.DMA,
        ],
        compiler_params=pltpu.CompilerParams(needs_layout_passes=False,
                                             use_tc_tiling_on_sc=True),
    )
    fac_lin = relayout(fac_view, tail_lin).reshape(NFEAT, LATENT)

    run = pl.kernel(
        _fm_body,
        out_type=jax.ShapeDtypeStruct((BATCH,), jnp.float32),
        mesh=mesh,
        scratch_types=[
            pltpu.VMEM((IDX_ROWS, GW), jnp.int32),
            pltpu.VMEM((CI, LATENT), jnp.float32),
            pltpu.VMEM((CI, LATENT), jnp.float32),
            pltpu.VMEM((CI + 16,), jnp.float32),
            pltpu.VMEM((CI + 16,), jnp.float32),
            pltpu.VMEM((RPW,), jnp.float32),
            pltpu.SemaphoreType.DMA,
            pltpu.SemaphoreType.DMA,
        ],
        compiler_params=pltpu.CompilerParams(needs_layout_passes=False,
                                             use_tc_tiling_on_sc=False),
    )
    out = run(x_idx, emb_bias_w.T, fac_lin)
    return out.reshape(BATCH, 1)


def kernel(x, emb_bias_w, emb_factor_w):
    return _fm(x, emb_bias_w, emb_factor_w)


# speedup vs baseline: 1.5874x; 1.0024x over previous
import jax
import jax.numpy as jnp
from jax import lax
from jax.experimental import pallas as pl
from jax.experimental.pallas import tpu as pltpu
from jax.experimental.pallas import tpu_sc as plsc

BATCH = 16384
FIELDS = 26
LATENT = 16
NFEAT = 1000000

NCORES = 2
NSUB = 16
NWORK = NCORES * NSUB

NTAIL64 = 64
VC = 1536
NFULL = (NFEAT - NTAIL64) // VC
ITERS1 = (NFULL + NWORK - 1) // NWORK

RPW = BATCH // NWORK
CR = 128
NCH = RPW // CR
CI = CR * FIELDS
GW = 128
NG = CI // GW
IDX_ROWS = RPW * FIELDS // GW


def _relayout_body(fac_hbm, tail_hbm, out_hbm, buf0, buf1, st0, st1, sem0, sem1):
    wid = lax.axis_index("s") * NCORES + lax.axis_index("c")
    lanes16 = lax.iota(jnp.int32, 16) * 16

    def issue(c, nv, buf, sem):
        pltpu.async_copy(fac_hbm.at[:, :, pl.ds(c * VC, nv)],
                         buf.at[:, :, pl.ds(0, nv)], sem)

    def drain(c, nv, buf, sem):
        pltpu.make_async_copy(fac_hbm.at[:, :, pl.ds(c * VC, nv)],
                              buf.at[:, :, pl.ds(0, nv)], sem).wait()

    def transpose(c, nv_blocks, buf, stage, sem):
        @pl.loop(0, nv_blocks)
        def _(b):
            base = lanes16 + b * (16 * LATENT)
            vals = [buf[a, s, pl.ds(b * 16, 16)]
                    for a in range(2) for s in range(8)]
            for k, v in enumerate(vals):
                plsc.store_scatter(stage, [base + k], v)
        pltpu.sync_copy(stage.at[pl.ds(0, nv_blocks * 16 * LATENT)],
                        out_hbm.at[pl.ds(c * VC * LATENT,
                                         nv_blocks * 16 * LATENT)])

    c0 = wid

    @pl.when(c0 < NFULL)
    def _():
        issue(c0, VC, buf0, sem0)

    @pl.loop(0, ITERS1, step=2)
    def _(it):
        ca = (it + 0) * NWORK + wid
        cb = (it + 1) * NWORK + wid
        cc = (it + 2) * NWORK + wid

        @pl.when(cb < NFULL)
        def _():
            issue(cb, VC, buf1, sem1)

        @pl.when(ca < NFULL)
        def _():
            drain(ca, VC, buf0, sem0)
            transpose(ca, VC // 16, buf0, st0, sem0)

        @pl.when(cc < NFULL)
        def _():
            issue(cc, VC, buf0, sem0)

        @pl.when(cb < NFULL)
        def _():
            drain(cb, VC, buf1, sem1)
            transpose(cb, VC // 16, buf1, st1, sem1)

    @pl.when(wid == NWORK - 2)
    def _():
        pltpu.sync_copy(tail_hbm, st0.at[pl.ds(0, NTAIL64 * LATENT)])
        pltpu.sync_copy(st0.at[pl.ds(0, NTAIL64 * LATENT)],
                        out_hbm.at[pl.ds((NFEAT - NTAIL64) * LATENT,
                                         NTAIL64 * LATENT)])


def _fm_body(x_hbm, bias_hbm, fac_hbm, out_hbm,
             idx_v, rows0, rows1, bias0, bias1, out_v, sem0, sem1):
    wid = lax.axis_index("s") * NCORES + lax.axis_index("c")

    pltpu.sync_copy(x_hbm.at[pl.ds(wid * IDX_ROWS, IDX_ROWS), :], idx_v)

    lanes = lax.iota(jnp.int32, 16)
    tail_mask = jnp.where(lanes < (FIELDS - 16), 1.0, 0.0).astype(jnp.float32)
    bias_1d = bias_hbm.at[0]

    def issue(ch, rows_v, bias_v, sem):
        for j in range(NG):
            irow = idx_v.at[ch * NG + j]
            pltpu.async_copy(fac_hbm.at[irow], rows_v.at[pl.ds(j * GW, GW), :], sem)
            pltpu.async_copy(bias_1d.at[irow],
                             bias_v.at[pl.ds(j * GW, GW)], sem)

    def drain(ch, rows_v, bias_v, sem):
        for j in range(NG):
            irow = idx_v.at[ch * NG + j]
            pltpu.make_async_copy(fac_hbm.at[irow],
                                  rows_v.at[pl.ds(j * GW, GW), :], sem).wait()
            pltpu.make_async_copy(bias_1d.at[irow],
                                  bias_v.at[pl.ds(j * GW, GW)], sem).wait()

    def compute(ch, rows_v, bias_v):
        @pl.loop(0, CR // 16)
        def _(g):
            def row_body(k, acc):
                base = (g * 16 + k) * FIELDS
                vs = [rows_v[base + f, :] for f in range(FIELDS)]
                ss = [vs[0], vs[1], vs[2], vs[3]]
                qs = [v * v for v in vs[:4]]
                for f in range(4, FIELDS):
                    ss[f % 4] = ss[f % 4] + vs[f]
                    qs[f % 4] = qs[f % 4] + vs[f] * vs[f]
                s = (ss[0] + ss[1]) + (ss[2] + ss[3])
                q = (qs[0] + qs[1]) + (qs[2] + qs[3])
                b1 = bias_v[pl.ds(base, 16)]
                b2 = bias_v[pl.ds(base + 16, 16)]
                tot = s * s - q + b1 + b2 * tail_mask
                return jnp.where(lanes == k, jnp.sum(tot), acc)

            acc = lax.fori_loop(0, 16, row_body,
                                jnp.zeros((16,), jnp.float32))
            out_v[pl.ds(ch * CR + g * 16, 16)] = acc

    issue(0, rows0, bias0, sem0)

    @pl.loop(0, NCH, step=2)
    def _(ch):
        issue(ch + 1, rows1, bias1, sem1)
        drain(ch, rows0, bias0, sem0)
        compute(ch, rows0, bias0)

        @pl.when(ch + 2 < NCH)
        def _():
            issue(ch + 2, rows0, bias0, sem0)

        drain(ch + 1, rows1, bias1, sem1)
        compute(ch + 1, rows1, bias1)

    pltpu.sync_copy(out_v, out_hbm.at[pl.ds(wid * RPW, RPW)])


@jax.jit
def _fm(x, emb_bias_w, emb_factor_w):
    x_idx = x.astype(jnp.int32).reshape(BATCH * FIELDS // GW, GW)
    fac_view = emb_factor_w.T.reshape(2, 8, NFEAT)
    tail_lin = emb_factor_w[NFEAT - NTAIL64:, :].reshape(NTAIL64 * LATENT)
    mesh = plsc.VectorSubcoreMesh(core_axis_name="c", subcore_axis_name="s")

    relayout = pl.kernel(
        _relayout_body,
        out_type=jax.ShapeDtypeStruct((NFEAT * LATENT,), jnp.float32),
        mesh=mesh,
        scratch_types=[
            pltpu.VMEM((2, 8, VC), jnp.float32),
            pltpu.VMEM((2, 8, VC), jnp.float32),
            pltpu.VMEM((VC * LATENT,), jnp.float32),
            pltpu.VMEM((VC * LATENT,), jnp.float32),
            pltpu.SemaphoreType.DMA,
            pltpu.SemaphoreType.DMA,
        ],
        compiler_params=pltpu.CompilerParams(needs_layout_passes=False,
                                             use_tc_tiling_on_sc=True),
    )
    fac_lin = relayout(fac_view, tail_lin).reshape(NFEAT, LATENT)

    run = pl.kernel(
        _fm_body,
        out_type=jax.ShapeDtypeStruct((BATCH,), jnp.float32),
        mesh=mesh,
        scratch_types=[
            pltpu.VMEM((IDX_ROWS, GW), jnp.int32),
            pltpu.VMEM((CI, LATENT), jnp.float32),
            pltpu.VMEM((CI, LATENT), jnp.float32),
            pltpu.VMEM((CI + 16,), jnp.float32),
            pltpu.VMEM((CI + 16,), jnp.float32),
            pltpu.VMEM((RPW,), jnp.float32),
            pltpu.SemaphoreType.DMA,
            pltpu.SemaphoreType.DMA,
        ],
        compiler_params=pltpu.CompilerParams(needs_layout_passes=False,
                                             use_tc_tiling_on_sc=False),
    )
    out = run(x_idx, emb_bias_w.T, fac_lin)
    return out.reshape(BATCH, 1)


def kernel(x, emb_bias_w, emb_factor_w):
    return _fm(x, emb_bias_w, emb_factor_w)
